# baseline (speedup 1.0000x reference)
import functools

import jax
import jax.numpy as jnp
from jax import lax
from jax.experimental import pallas as pl
from jax.experimental.pallas import tpu as pltpu
from jax.experimental.pallas import tpu_sc as plsc

B = 16384
NV = 100
VSPLIT = ((0, 24, 8), (24, 24, 8), (48, 24, 8), (72, 28, 7))
NG = 1000
NW = 32
RW = B // NW
QR = 4 * RW
CH = 128
NCH = QR // CH
L = 16


def _sc_body(gene_hbm, table_hbm, out_hbm,
             table_v, g0a_v, g1a_v, g0b_v, g1b_v, part_v, shared_s,
             sem_t, sem0, sem1, sem_x):
    cid = lax.axis_index("c")
    sid = lax.axis_index("s")
    quad = sid // 4 * 4
    role = sid % 4

    fzero = jnp.zeros((L,), jnp.float32)

    def worker_base(i):
        return ((quad + i) * 2 + cid) * RW

    def chunk_row(c):
        return worker_base(c // 4) + (c % 4) * CH

    def compute(v0, nv, ub, bufs, sems):
        tbl_cp = pltpu.make_async_copy(
            table_hbm.at[pl.ds(v0 * NG, nv * NG)],
            table_v.at[pl.ds(0, nv * NG)], sem_t)
        tbl_cp.start()

        def gene_copy(c):
            return pltpu.make_async_copy(
                gene_hbm.at[pl.ds(v0, nv), pl.ds(chunk_row(c), CH)],
                bufs[c % 2], sems[c % 2])

        cp = gene_copy(0)
        cp.start()
        tbl_cp.wait()

        for c in range(NCH):
            cp.wait()
            if c + 1 < NCH:
                cp = gene_copy(c + 1)
                cp.start()
            gbuf = bufs[c % 2]

            def group_body(i0, _):
                col = i0 * L

                def blk(b, carry):
                    acc, prev = carry
                    new = []
                    for j in range(ub):
                        v = b * ub + j
                        g = gbuf[v, pl.ds(col, L)]
                        new.append(
                            plsc.load_gather(table_v, [g + v * NG]))
                    for x in prev:
                        acc = acc + x
                    return acc, tuple(new)

                acc, last = lax.fori_loop(0, nv // ub, blk,
                                          (fzero, (fzero,) * ub))
                for x in last:
                    acc = acc + x
                part_v[pl.ds(c * CH + col, L)] = acc
                return 0

            lax.fori_loop(0, CH // L, group_body, 0)

    for r in range(4):
        @pl.when(role == r)
        def _branch(r=r):
            v0, nv, ub = VSPLIT[r]
            bufs = (g0a_v, g1a_v) if r < 3 else (g0b_v, g1b_v)
            compute(v0, nv, ub, bufs, (sem0, sem1))

    for i in range(4):
        @pl.when(role != i)
        def _publish(i=i):
            sid_p = quad + i
            pltpu.sync_copy(
                part_v.at[pl.ds(i * RW, RW)],
                shared_s.at[pl.ds((sid_p * 4 + role) * RW, RW)])

    plsc.subcore_barrier()

    own_off = role * RW
    for k in range(4):
        @pl.when(role != k)
        def _fetch(k=k):
            xcp = pltpu.make_async_copy(
                shared_s.at[pl.ds((sid * 4 + k) * RW, RW)],
                part_v.at[pl.ds(RW * 4 + k * RW, RW)], sem_x)
            xcp.start()
            xcp.wait()

    def add_body(i, _):
        off = i * L
        acc = part_v[pl.ds(own_off + off, L)]
        for k in range(4):
            acc = acc + jnp.where(
                role == k, fzero,
                part_v[pl.ds(RW * 4 + k * RW + off, L)])
        part_v[pl.ds(own_off + off, L)] = acc
        return 0

    lax.fori_loop(0, RW // L, add_body, 0)

    pltpu.sync_copy(part_v.at[pl.ds(own_off, RW)],
                    out_hbm.at[pl.ds(worker_base(role), RW)])


@jax.jit
def kernel(gene, genes):
    gene_t = gene.astype(jnp.int32).T
    table_flat = genes.reshape(-1).astype(jnp.float32)

    sc_call = functools.partial(
        pl.kernel,
        mesh=plsc.VectorSubcoreMesh(core_axis_name="c", subcore_axis_name="s"),
        out_type=jax.ShapeDtypeStruct((B,), jnp.float32),
        scratch_types=[
            pltpu.VMEM((28 * NG,), jnp.float32),
            pltpu.VMEM((24, CH), jnp.int32),
            pltpu.VMEM((24, CH), jnp.int32),
            pltpu.VMEM((28, CH), jnp.int32),
            pltpu.VMEM((28, CH), jnp.int32),
            pltpu.VMEM((8 * RW,), jnp.float32),
            pltpu.VMEM_SHARED((16 * 4 * RW,), jnp.float32),
            pltpu.SemaphoreType.DMA,
            pltpu.SemaphoreType.DMA,
            pltpu.SemaphoreType.DMA,
            pltpu.SemaphoreType.DMA,
        ],
        compiler_params=pltpu.CompilerParams(needs_layout_passes=False),
    )(_sc_body)

    fit = sc_call(gene_t, table_flat)
    return fit.reshape(B, 1)


# speedup vs baseline: 1.2925x; 1.2925x over previous
import functools

import jax
import jax.numpy as jnp
from jax import lax
from jax.experimental import pallas as pl
from jax.experimental.pallas import tpu as pltpu
from jax.experimental.pallas import tpu_sc as plsc

B = 16384
NV = 100
NVA = 48
NVB = NV - NVA
NG = 1000
NW = 32
RW = B // NW
PR = 2 * RW
CH = 256
NCH = PR // CH
L = 16


def _sc_body(gene_hbm, table_hbm, out_hbm,
             table_v, g0a_v, g1a_v, g0b_v, g1b_v, part_v, shared_s,
             sem_t, sem0, sem1, sem_x):
    cid = lax.axis_index("c")
    sid = lax.axis_index("s")
    wid = sid * 2 + cid
    wid_p = (sid ^ 1) * 2 + cid
    base_own = wid * RW
    base_par = wid_p * RW

    fzero = jnp.zeros((L,), jnp.float32)

    def chunk_row(c):
        if c < NCH // 2:
            return base_own + c * CH
        return base_par + (c - NCH // 2) * CH

    def compute(v0, nv, ub, bufs, sems):
        tbl_cp = pltpu.make_async_copy(
            table_hbm.at[pl.ds(v0 * NG, nv * NG)],
            table_v.at[pl.ds(0, nv * NG)], sem_t)
        tbl_cp.start()

        def gene_copy(c):
            return pltpu.make_async_copy(
                gene_hbm.at[pl.ds(v0, nv), pl.ds(chunk_row(c), CH)],
                bufs[c % 2], sems[c % 2])

        cp = gene_copy(0)
        cp.start()
        tbl_cp.wait()

        for c in range(NCH):
            cp.wait()
            if c + 1 < NCH:
                cp = gene_copy(c + 1)
                cp.start()
            gbuf = bufs[c % 2]

            def group_body(i0, _):
                col = i0 * L

                def blk(b, carry):
                    acc, prev = carry
                    new = []
                    for j in range(ub):
                        v = b * ub + j
                        g = gbuf[v, pl.ds(col, L)]
                        new.append(
                            plsc.load_gather(table_v, [g + v * NG]))
                    for x in prev:
                        acc = acc + x
                    return acc, tuple(new)

                acc, last = lax.fori_loop(0, nv // ub, blk,
                                          (fzero, (fzero,) * ub))
                for x in last:
                    acc = acc + x
                part_v[pl.ds(c * CH + col, L)] = acc
                return 0

            lax.fori_loop(0, CH // L, group_body, 0)

    @pl.when(sid % 2 == 0)
    def _even():
        compute(0, NVA, 8, (g0a_v, g1a_v), (sem0, sem1))

    @pl.when(sid % 2 == 1)
    def _odd():
        compute(NVA, NVB, 13, (g0b_v, g1b_v), (sem0, sem1))

    sid_p = sid ^ 1
    pltpu.sync_copy(part_v.at[pl.ds(RW, RW)],
                    shared_s.at[pl.ds(sid_p * RW, RW)])
    plsc.subcore_barrier()
    xcp = pltpu.make_async_copy(shared_s.at[pl.ds(sid * RW, RW)],
                                part_v.at[pl.ds(RW, RW)], sem_x)
    xcp.start()
    xcp.wait()

    def add_body(i, _):
        off = i * L
        part_v[pl.ds(off, L)] = (part_v[pl.ds(off, L)]
                                 + part_v[pl.ds(RW + off, L)])
        return 0

    lax.fori_loop(0, RW // L, add_body, 0)

    pltpu.sync_copy(part_v.at[pl.ds(0, RW)],
                    out_hbm.at[pl.ds(base_own, RW)])


@jax.jit
def kernel(gene, genes):
    gene_t = gene.astype(jnp.int32).T
    table_flat = genes.reshape(-1).astype(jnp.float32)

    sc_call = functools.partial(
        pl.kernel,
        mesh=plsc.VectorSubcoreMesh(core_axis_name="c", subcore_axis_name="s"),
        out_type=jax.ShapeDtypeStruct((B,), jnp.float32),
        scratch_types=[
            pltpu.VMEM((NVB * NG,), jnp.float32),
            pltpu.VMEM((NVA, CH), jnp.int32),
            pltpu.VMEM((NVA, CH), jnp.int32),
            pltpu.VMEM((NVB, CH), jnp.int32),
            pltpu.VMEM((NVB, CH), jnp.int32),
            pltpu.VMEM((PR,), jnp.float32),
            pltpu.VMEM_SHARED((16 * RW,), jnp.float32),
            pltpu.SemaphoreType.DMA,
            pltpu.SemaphoreType.DMA,
            pltpu.SemaphoreType.DMA,
            pltpu.SemaphoreType.DMA,
        ],
        compiler_params=pltpu.CompilerParams(needs_layout_passes=False),
    )(_sc_body)

    fit = sc_call(gene_t, table_flat)
    return fit.reshape(B, 1)


# speedup vs baseline: 1.2970x; 1.0035x over previous
import functools

import jax
import jax.numpy as jnp
from jax import lax
from jax.experimental import pallas as pl
from jax.experimental.pallas import tpu as pltpu
from jax.experimental.pallas import tpu_sc as plsc

B = 16384
NV = 100
NVA = 48
NVB = NV - NVA
NG = 1000
NW = 32
RW = B // NW
PR = 2 * RW
CH = 256
NCH = PR // CH
L = 16


def _sc_body(gene_hbm, table_hbm, out_hbm,
             table_v, g0a_v, g1a_v, g0b_v, g1b_v, part_v, shared_s,
             sem_t, sem0, sem1, sem_x):
    cid = lax.axis_index("c")
    sid = lax.axis_index("s")
    wid = sid * 2 + cid
    sid_p = sid ^ 1
    wid_p = sid_p * 2 + cid
    base_own = wid * RW
    base_par = wid_p * RW

    fzero = jnp.zeros((L,), jnp.float32)

    def chunk_row(c):
        if c < NCH // 2:
            return base_par + c * CH
        return base_own + (c - NCH // 2) * CH

    def compute(v0, nv, ub, bufs, sems):
        tbl_cp = pltpu.make_async_copy(
            table_hbm.at[pl.ds(v0 * NG, nv * NG)],
            table_v.at[pl.ds(0, nv * NG)], sem_t)
        tbl_cp.start()

        def gene_copy(c):
            return pltpu.make_async_copy(
                gene_hbm.at[pl.ds(v0, nv), pl.ds(chunk_row(c), CH)],
                bufs[c % 2], sems[c % 2])

        cp = gene_copy(0)
        cp.start()
        tbl_cp.wait()

        for c in range(NCH):
            if c == NCH // 2:
                pltpu.sync_copy(part_v.at[pl.ds(0, RW)],
                                shared_s.at[pl.ds(sid_p * RW, RW)])
            cp.wait()
            if c + 1 < NCH:
                cp = gene_copy(c + 1)
                cp.start()
            gbuf = bufs[c % 2]

            def group_body(i0, _):
                col = i0 * L

                def blk(b, carry):
                    acc, prev = carry
                    new = []
                    for j in range(ub):
                        v = b * ub + j
                        g = gbuf[v, pl.ds(col, L)]
                        new.append(
                            plsc.load_gather(table_v, [g + v * NG]))
                    for x in prev:
                        acc = acc + x
                    return acc, tuple(new)

                acc, last = lax.fori_loop(0, nv // ub, blk,
                                          (fzero, (fzero,) * ub))
                for x in last:
                    acc = acc + x
                part_v[pl.ds(c * CH + col, L)] = acc
                return 0

            lax.fori_loop(0, CH // L, group_body, 0)

    @pl.when(sid % 2 == 0)
    def _even():
        compute(0, NVA, 8, (g0a_v, g1a_v), (sem0, sem1))

    @pl.when(sid % 2 == 1)
    def _odd():
        compute(NVA, NVB, 13, (g0b_v, g1b_v), (sem0, sem1))

    plsc.subcore_barrier()
    xcp = pltpu.make_async_copy(shared_s.at[pl.ds(sid * RW, RW)],
                                part_v.at[pl.ds(0, RW)], sem_x)
    xcp.start()
    xcp.wait()

    def add_body(i, _):
        off = i * L
        part_v[pl.ds(off, L)] = (part_v[pl.ds(off, L)]
                                 + part_v[pl.ds(RW + off, L)])
        return 0

    lax.fori_loop(0, RW // L, add_body, 0)

    pltpu.sync_copy(part_v.at[pl.ds(0, RW)],
                    out_hbm.at[pl.ds(base_own, RW)])


@jax.jit
def kernel(gene, genes):
    gene_t = gene.astype(jnp.int32).T
    table_flat = genes.reshape(-1).astype(jnp.float32)

    sc_call = functools.partial(
        pl.kernel,
        mesh=plsc.VectorSubcoreMesh(core_axis_name="c", subcore_axis_name="s"),
        out_type=jax.ShapeDtypeStruct((B,), jnp.float32),
        scratch_types=[
            pltpu.VMEM((NVB * NG,), jnp.float32),
            pltpu.VMEM((NVA, CH), jnp.int32),
            pltpu.VMEM((NVA, CH), jnp.int32),
            pltpu.VMEM((NVB, CH), jnp.int32),
            pltpu.VMEM((NVB, CH), jnp.int32),
            pltpu.VMEM((PR,), jnp.float32),
            pltpu.VMEM_SHARED((16 * RW,), jnp.float32),
            pltpu.SemaphoreType.DMA,
            pltpu.SemaphoreType.DMA,
            pltpu.SemaphoreType.DMA,
            pltpu.SemaphoreType.DMA,
        ],
        compiler_params=pltpu.CompilerParams(needs_layout_passes=False),
    )(_sc_body)

    fit = sc_call(gene_t, table_flat)
    return fit.reshape(B, 1)
